# baseline (speedup 1.0000x reference)
import functools

import jax
import jax.numpy as jnp
from jax import lax
from jax.experimental import pallas as pl
from jax.experimental.pallas import tpu as pltpu, tpu_sc as plsc

_INFO = plsc.get_sparse_core_info()
_NC = _INFO.num_cores
_NS = _INFO.num_subcores
_NW = _NC * _NS
_CHUNK = 128
_L = 16


@functools.partial(jax.jit, static_argnums=(2, 3))
def _gather_t(idx, w_all, b_per_w, d):
    n_chunks = b_per_w // _CHUNK
    batch = idx.shape[0]
    mesh = plsc.VectorSubcoreMesh(core_axis_name="c", subcore_axis_name="s")

    @functools.partial(
        pl.kernel,
        mesh=mesh,
        out_type=jax.ShapeDtypeStruct((d, batch), jnp.float32),
        scratch_types=[
            pltpu.VMEM((b_per_w,), jnp.int32),
            pltpu.VMEM((b_per_w, d), jnp.float32),
            pltpu.VMEM((d, b_per_w + 1), jnp.float32),
            pltpu.SemaphoreType.DMA((4,)),
            pltpu.SemaphoreType.DMA,
        ],
        compiler_params=pltpu.CompilerParams(
            use_tc_tiling_on_sc=False, needs_layout_passes=False),
    )
    def body(table_hbm, idx_hbm, out_hbm, idx_v, rows_v, cols_v, gsem, wsem):
        wid = lax.axis_index("s") * _NC + lax.axis_index("c")
        base = wid * b_per_w
        pltpu.sync_copy(idx_hbm.at[pl.ds(base, b_per_w)], idx_v)
        gathers = [
            pltpu.make_async_copy(
                table_hbm.at[idx_v.at[pl.ds(j * _CHUNK, _CHUNK)]],
                rows_v.at[pl.ds(j * _CHUNK, _CHUNK)],
                gsem.at[j % 4],
            )
            for j in range(n_chunks)
        ]
        for c in gathers:
            c.start()

        lane = lax.iota(jnp.int32, _L)
        dvecs = [lane + db * _L for db in range(d // _L)]

        def tbody(i, carry):
            for u in range(4):
                bi = i * 4 + u
                ci = jnp.full((_L,), 0, jnp.int32) + bi
                for db in range(d // _L):
                    v = rows_v[bi, pl.ds(db * _L, _L)]
                    plsc.store_scatter(cols_v, [dvecs[db], ci], v)
            return carry

        writes = []
        for j in range(n_chunks):
            gathers[j].wait()
            lax.fori_loop(j * (_CHUNK // 4), (j + 1) * (_CHUNK // 4),
                          tbody, 0)
            w = pltpu.make_async_copy(
                cols_v.at[:, pl.ds(j * _CHUNK, _CHUNK)],
                out_hbm.at[:, pl.ds(base + j * _CHUNK, _CHUNK)], wsem)
            w.start()
            writes.append(w)
        for w in writes:
            w.wait()

    return body(w_all, idx)


def kernel(inputs, w_all):
    batch = inputs.shape[0]
    d = w_all.shape[1]
    b_per_w = batch // _NW
    out_t = _gather_t(inputs.astype(jnp.int32), w_all.astype(jnp.float32),
                      b_per_w, d)
    return out_t.T[:, :, None]


# speedup vs baseline: 1.1838x; 1.1838x over previous
import functools

import jax
import jax.numpy as jnp
from jax import lax
from jax.experimental import pallas as pl
from jax.experimental.pallas import tpu as pltpu, tpu_sc as plsc

_INFO = plsc.get_sparse_core_info()
_NC = _INFO.num_cores
_NS = _INFO.num_subcores
_NW = _NC * _NS
_CHUNK = 128
_L = 16


@functools.partial(jax.jit, static_argnums=(2, 3))
def _gather_t(idx, w_all, b_per_w, d):
    n_chunks = b_per_w // _CHUNK
    batch = idx.shape[0]
    mesh = plsc.VectorSubcoreMesh(core_axis_name="c", subcore_axis_name="s")

    @functools.partial(
        pl.kernel,
        mesh=mesh,
        out_type=jax.ShapeDtypeStruct((d, batch), jnp.float32),
        scratch_types=[
            pltpu.VMEM((b_per_w,), jnp.int32),
            pltpu.VMEM((b_per_w, d), jnp.float32),
            pltpu.VMEM((d, b_per_w + 1), jnp.float32),
            pltpu.SemaphoreType.DMA((4,)),
            pltpu.SemaphoreType.DMA,
        ],
        compiler_params=pltpu.CompilerParams(
            use_tc_tiling_on_sc=False, needs_layout_passes=False),
    )
    def body(table_hbm, idx_hbm, out_hbm, idx_v, rows_v, cols_v, gsem, wsem):
        wid = lax.axis_index("s") * _NC + lax.axis_index("c")
        base = wid * b_per_w
        pltpu.sync_copy(idx_hbm.at[pl.ds(base, b_per_w)], idx_v)
        gathers = [
            pltpu.make_async_copy(
                table_hbm.at[idx_v.at[pl.ds(j * _CHUNK, _CHUNK)]],
                rows_v.at[pl.ds(j * _CHUNK, _CHUNK)],
                gsem.at[j % 4],
            )
            for j in range(n_chunks)
        ]
        for c in gathers:
            c.start()
        for c in gathers:
            c.wait()

        lane = lax.iota(jnp.int32, _L)
        dvecs = [lane + db * _L for db in range(d // _L)]

        @plsc.parallel_loop(0, b_per_w, 4, unroll=2)
        def tbody(i):
            for u in range(4):
                bi = i + u
                ci = jnp.full((_L,), 0, jnp.int32) + bi
                for db in range(d // _L):
                    v = rows_v[bi, pl.ds(db * _L, _L)]
                    plsc.store_scatter(cols_v, [dvecs[db], ci], v)

        w = pltpu.make_async_copy(
            cols_v.at[:, pl.ds(0, b_per_w)],
            out_hbm.at[:, pl.ds(base, b_per_w)], wsem)
        w.start()
        w.wait()

    return body(w_all, idx)


def kernel(inputs, w_all):
    batch = inputs.shape[0]
    d = w_all.shape[1]
    b_per_w = batch // _NW
    out_t = _gather_t(inputs.astype(jnp.int32), w_all.astype(jnp.float32),
                      b_per_w, d)
    return out_t.T[:, :, None]
